# Initial kernel scaffold; baseline (speedup 1.0000x reference)
import jax
import jax.numpy as jnp
from jax.experimental import pallas as pl


def kernel(x, weight):
    raise NotImplementedError("write your pallas kernel here")


# speedup vs baseline: 1.0789x; 1.0789x over previous
import functools

import jax
import jax.numpy as jnp
from jax import lax
from jax.experimental import pallas as pl
from jax.experimental.pallas import tpu as pltpu
from jax.experimental.pallas import tpu_sc as plsc

NUM_CORES = 2
NUM_SUBCORES = 16
NUM_WORKERS = NUM_CORES * NUM_SUBCORES

ROWS_PER_GATHER = 128
GATHERS_PER_CHUNK = 8
CHUNK = ROWS_PER_GATHER * GATHERS_PER_CHUNK


@functools.lru_cache(maxsize=None)
def _make_kernel(B, V, D):
    assert B % (NUM_WORKERS * CHUNK) == 0
    b_per_w = B // NUM_WORKERS
    nchunks = b_per_w // CHUNK
    groups_pw = b_per_w // ROWS_PER_GATHER

    mesh = plsc.VectorSubcoreMesh(core_axis_name="c", subcore_axis_name="s")

    @functools.partial(
        pl.kernel,
        mesh=mesh,
        compiler_params=pltpu.CompilerParams(use_tc_tiling_on_sc=False),
        out_type=jax.ShapeDtypeStruct((B, D), jnp.float32),
        scratch_types=[
            pltpu.VMEM((groups_pw, ROWS_PER_GATHER), jnp.int32),
            pltpu.VMEM((CHUNK, D), jnp.float32),
            pltpu.SemaphoreType.DMA,
        ],
    )
    def emb(table_hbm, idx_hbm, out_hbm, idx_v, rows_v, gsem):
        wid = lax.axis_index("s") * NUM_CORES + lax.axis_index("c")
        gbase = wid * groups_pw
        rbase = wid * b_per_w
        pltpu.sync_copy(idx_hbm.at[pl.ds(gbase, groups_pw)], idx_v)

        def body(g, carry):
            descs = []
            for j in range(GATHERS_PER_CHUNK):
                d = pltpu.async_copy(
                    table_hbm.at[idx_v.at[g * GATHERS_PER_CHUNK + j]],
                    rows_v.at[pl.ds(j * ROWS_PER_GATHER, ROWS_PER_GATHER)],
                    gsem,
                )
                descs.append(d)
            for d in descs:
                d.wait()
            pltpu.sync_copy(rows_v, out_hbm.at[pl.ds(rbase + g * CHUNK, CHUNK)])
            return carry

        lax.fori_loop(0, nchunks, body, 0)

    return emb


def kernel(x, weight):
    B = x.size
    D = weight.shape[1]
    idx = x.reshape(B // ROWS_PER_GATHER, ROWS_PER_GATHER).astype(jnp.int32)
    emb = _make_kernel(B, weight.shape[0], D)
    out = emb(weight, idx)
    return out.reshape(B, 1, D)
